# Initial kernel scaffold; baseline (speedup 1.0000x reference)
import jax
import jax.numpy as jnp
from jax.experimental import pallas as pl


def kernel(x1, adj, W1, b1, W2, b2):
    raise NotImplementedError("write your pallas kernel here")


# speedup vs baseline: 1.4623x; 1.4623x over previous
import functools

import jax
import jax.numpy as jnp
from jax import lax
from jax.experimental import pallas as pl
from jax.experimental.pallas import tpu as pltpu
from jax.experimental.pallas import tpu_sc as plsc

N_NODES = 10000
N_CH = 128
CPT = 4
NTILES = 32
NC = 2
FLAT = N_NODES * CPT
E_TOTAL = 320000
CHUNK = 4000
NBATCH = CHUNK // 16
NCHUNK = E_TOTAL // CHUNK


def _sc_gather_segmax(xt, src_e, dst_e):
    mesh = plsc.VectorSubcoreMesh(core_axis_name="c", subcore_axis_name="s")

    @functools.partial(
        pl.kernel,
        out_type=jax.ShapeDtypeStruct((NTILES, FLAT), jnp.float32),
        mesh=mesh,
        scratch_types=[
            pltpu.VMEM((FLAT,), jnp.float32),
            pltpu.VMEM((FLAT,), jnp.float32),
            pltpu.VMEM((CHUNK,), jnp.int32),
            pltpu.VMEM((CHUNK,), jnp.int32),
        ],
        compiler_params=pltpu.CompilerParams(needs_layout_passes=False),
    )
    def k(xt_hbm, src_hbm, dst_hbm, out_hbm, xc_v, agg_v, src_v, dst_v):
        wid = lax.axis_index("s") * NC + lax.axis_index("c")
        pltpu.sync_copy(xt_hbm.at[wid], xc_v)

        neg_inf = jnp.full((16,), -jnp.inf, dtype=jnp.float32)

        def init_body(i, carry):
            agg_v[pl.ds(i * 16, 16)] = neg_inf
            return carry

        lax.fori_loop(0, FLAT // 16, init_body, 0)

        iota = lax.iota(jnp.int32, 16)

        def chunk_body(ci, carry):
            e0 = ci * CHUNK
            pltpu.sync_copy(src_hbm.at[pl.ds(e0, CHUNK)], src_v)
            pltpu.sync_copy(dst_hbm.at[pl.ds(e0, CHUNK)], dst_v)

            def batch_body(b, c2):
                src = src_v[pl.ds(b * 16, 16)]
                dst = dst_v[pl.ds(b * 16, 16)]
                src4 = src * CPT
                dst4 = dst * CPT

                xv = []
                for ch in range(CPT):
                    xv.append(plsc.load_gather(xc_v, [src4 + ch]))
                    old = plsc.load_gather(agg_v, [dst4 + ch])
                    plsc.store_scatter(
                        agg_v, [dst4 + ch], jnp.maximum(old, xv[ch])
                    )

                _, lastm = plsc.scan_count(dst)
                ndup = jnp.max(jnp.where(lastm, 0, 1))

                @pl.when(ndup > 0)
                def _slow():
                    for lane in range(16):
                        m = iota == lane
                        for ch in range(CPT):
                            o = plsc.load_gather(agg_v, [dst4 + ch])
                            plsc.store_scatter(
                                agg_v,
                                [dst4 + ch],
                                jnp.maximum(o, xv[ch]),
                                mask=m,
                            )

                return c2

            lax.fori_loop(0, NBATCH, batch_body, 0)
            return carry

        lax.fori_loop(0, NCHUNK, chunk_body, 0)

        def fin_body(i, carry):
            sl = pl.ds(i * 16, 16)
            a = agg_v[sl]
            agg_v[sl] = jnp.where(a == -jnp.inf, 0.0, a) + xc_v[sl]
            return carry

        lax.fori_loop(0, FLAT // 16, fin_body, 0)
        pltpu.sync_copy(agg_v, out_hbm.at[wid])

    return k(xt, src_e, dst_e)


def _tc_mlp(h, W1, b1, W2, b2):
    BLK = 1000

    def mlp_body(h_ref, w1_ref, b1_ref, w2_ref, b2_ref, o_ref):
        hh = h_ref[...]
        z = jnp.dot(hh, w1_ref[...], preferred_element_type=jnp.float32)
        z = jnp.maximum(z + b1_ref[...], 0.0)
        o = jnp.dot(z, w2_ref[...], preferred_element_type=jnp.float32)
        o_ref[...] = o + b2_ref[...]

    return pl.pallas_call(
        mlp_body,
        grid=(N_NODES // BLK,),
        in_specs=[
            pl.BlockSpec((BLK, N_CH), lambda i: (i, 0)),
            pl.BlockSpec((N_CH, N_CH), lambda i: (0, 0)),
            pl.BlockSpec((1, N_CH), lambda i: (0, 0)),
            pl.BlockSpec((N_CH, N_CH), lambda i: (0, 0)),
            pl.BlockSpec((1, N_CH), lambda i: (0, 0)),
        ],
        out_specs=pl.BlockSpec((BLK, N_CH), lambda i: (i, 0)),
        out_shape=jax.ShapeDtypeStruct((N_NODES, N_CH), jnp.float32),
    )(h, W1, b1.reshape(1, N_CH), W2, b2.reshape(1, N_CH))


def kernel(x1, adj, W1, b1, W2, b2):
    xt = x1.reshape(N_NODES, NTILES, CPT).transpose(1, 0, 2).reshape(NTILES, FLAT)
    h32 = _sc_gather_segmax(xt, adj[0], adj[1])
    h = h32.reshape(NTILES, N_NODES, CPT).transpose(1, 0, 2).reshape(N_NODES, N_CH)
    return _tc_mlp(h, W1, b1, W2, b2)


# speedup vs baseline: 1.6630x; 1.1372x over previous
import functools

import jax
import jax.numpy as jnp
from jax import lax
from jax.experimental import pallas as pl
from jax.experimental.pallas import tpu as pltpu
from jax.experimental.pallas import tpu_sc as plsc

N_NODES = 10000
N_CH = 128
CPT = 4
NTILES = 32
NC = 2
FLAT = N_NODES * CPT
E_TOTAL = 320000
CHUNK = 6400
SUPER = 4
NSUPER = CHUNK // (16 * SUPER)
NCHUNK = E_TOTAL // CHUNK


def _sc_gather_segmax(xt, src_e, dst_e):
    mesh = plsc.VectorSubcoreMesh(core_axis_name="c", subcore_axis_name="s")

    @functools.partial(
        pl.kernel,
        out_type=jax.ShapeDtypeStruct((NTILES, FLAT), jnp.float32),
        mesh=mesh,
        scratch_types=[
            pltpu.VMEM((FLAT,), jnp.float32),
            pltpu.VMEM((FLAT,), jnp.float32),
            pltpu.VMEM((N_NODES,), jnp.int32),
            pltpu.VMEM((CHUNK,), jnp.int32),
            pltpu.VMEM((CHUNK,), jnp.int32),
        ],
        compiler_params=pltpu.CompilerParams(needs_layout_passes=False),
    )
    def k(xt_hbm, src_hbm, dst_hbm, out_hbm, xc_v, agg_v, claim_v, src_v, dst_v):
        wid = lax.axis_index("s") * NC + lax.axis_index("c")
        pltpu.sync_copy(xt_hbm.at[wid], xc_v)

        neg_inf = jnp.full((16,), -jnp.inf, dtype=jnp.float32)

        def init_body(i, carry):
            agg_v[pl.ds(i * 16, 16)] = neg_inf
            return carry

        lax.fori_loop(0, FLAT // 16, init_body, 0)

        iota = lax.iota(jnp.int32, 16)

        def chunk_body(ci, carry):
            e0 = ci * CHUNK
            pltpu.sync_copy(src_hbm.at[pl.ds(e0, CHUNK)], src_v)
            pltpu.sync_copy(dst_hbm.at[pl.ds(e0, CHUNK)], dst_v)

            def super_body(sb, c2):
                infos = []
                for u in range(SUPER):
                    off = (sb * SUPER + u) * 16
                    src = src_v[pl.ds(off, 16)]
                    dst = dst_v[pl.ds(off, 16)]
                    src4 = src * CPT
                    dst4 = dst * CPT
                    plsc.store_scatter(claim_v, [dst], iota)
                    w = plsc.load_gather(claim_v, [dst])
                    dup = w != iota
                    xvs = []
                    for ch in range(CPT):
                        xv = plsc.load_gather(xc_v, [src4 + ch])
                        xvs.append(xv)
                        old = plsc.load_gather(agg_v, [dst4 + ch])
                        plsc.store_scatter(agg_v, [dst4 + ch], jnp.maximum(old, xv))
                    infos.append((dst4, xvs, dup))

                anyd = infos[0][2]
                for u in range(1, SUPER):
                    anyd = anyd | infos[u][2]

                @pl.when(jnp.max(anyd.astype(jnp.int32)) > 0)
                def _slow():
                    for dst4, xvs, dup in infos:
                        @pl.when(jnp.max(dup.astype(jnp.int32)) > 0)
                        def _fix(dst4=dst4, xvs=xvs):
                            for lane in range(16):
                                m = iota == lane
                                for ch in range(CPT):
                                    o = plsc.load_gather(agg_v, [dst4 + ch])
                                    plsc.store_scatter(
                                        agg_v,
                                        [dst4 + ch],
                                        jnp.maximum(o, xvs[ch]),
                                        mask=m,
                                    )

                return c2

            lax.fori_loop(0, NSUPER, super_body, 0)
            return carry

        lax.fori_loop(0, NCHUNK, chunk_body, 0)

        def fin_body(i, carry):
            sl = pl.ds(i * 16, 16)
            a = agg_v[sl]
            agg_v[sl] = jnp.where(a == -jnp.inf, 0.0, a) + xc_v[sl]
            return carry

        lax.fori_loop(0, FLAT // 16, fin_body, 0)
        pltpu.sync_copy(agg_v, out_hbm.at[wid])

    return k(xt, src_e, dst_e)


def _tc_mlp(h, W1, b1, W2, b2):
    BLK = 1000

    def mlp_body(h_ref, w1_ref, b1_ref, w2_ref, b2_ref, o_ref):
        hh = h_ref[...]
        z = jnp.dot(hh, w1_ref[...], preferred_element_type=jnp.float32)
        z = jnp.maximum(z + b1_ref[...], 0.0)
        o = jnp.dot(z, w2_ref[...], preferred_element_type=jnp.float32)
        o_ref[...] = o + b2_ref[...]

    return pl.pallas_call(
        mlp_body,
        grid=(N_NODES // BLK,),
        in_specs=[
            pl.BlockSpec((BLK, N_CH), lambda i: (i, 0)),
            pl.BlockSpec((N_CH, N_CH), lambda i: (0, 0)),
            pl.BlockSpec((1, N_CH), lambda i: (0, 0)),
            pl.BlockSpec((N_CH, N_CH), lambda i: (0, 0)),
            pl.BlockSpec((1, N_CH), lambda i: (0, 0)),
        ],
        out_specs=pl.BlockSpec((BLK, N_CH), lambda i: (i, 0)),
        out_shape=jax.ShapeDtypeStruct((N_NODES, N_CH), jnp.float32),
    )(h, W1, b1.reshape(1, N_CH), W2, b2.reshape(1, N_CH))


def kernel(x1, adj, W1, b1, W2, b2):
    xt = x1.reshape(N_NODES, NTILES, CPT).transpose(1, 0, 2).reshape(NTILES, FLAT)
    h32 = _sc_gather_segmax(xt, adj[0], adj[1])
    h = h32.reshape(NTILES, N_NODES, CPT).transpose(1, 0, 2).reshape(N_NODES, N_CH)
    return _tc_mlp(h, W1, b1, W2, b2)


# speedup vs baseline: 2.2930x; 1.3788x over previous
import functools

import jax
import jax.numpy as jnp
from jax import lax
from jax.experimental import pallas as pl
from jax.experimental.pallas import tpu as pltpu
from jax.experimental.pallas import tpu_sc as plsc

N_NODES = 10000
N_CH = 128
CPT = 4
NTILES = 32
NC = 2
FLAT = N_NODES * CPT
E_TOTAL = 320000
CHUNK = 6400
SUPER = 4
NSUPER = CHUNK // (16 * SUPER)
NCHUNK = E_TOTAL // CHUNK


def _sc_gather_segmax(xt, src_e, dst_e):
    mesh = plsc.VectorSubcoreMesh(core_axis_name="c", subcore_axis_name="s")

    @functools.partial(
        pl.kernel,
        out_type=jax.ShapeDtypeStruct((NTILES, FLAT), jnp.float32),
        mesh=mesh,
        scratch_types=[
            pltpu.VMEM((FLAT,), jnp.float32),
            pltpu.VMEM((FLAT,), jnp.float32),
            pltpu.VMEM((N_NODES,), jnp.int32),
            pltpu.VMEM((CHUNK,), jnp.int32),
            pltpu.VMEM((CHUNK,), jnp.int32),
        ],
        compiler_params=pltpu.CompilerParams(needs_layout_passes=False),
    )
    def k(xt_hbm, src_hbm, dst_hbm, out_hbm, xc_v, agg_v, claim_v, src_v, dst_v):
        wid = lax.axis_index("s") * NC + lax.axis_index("c")
        pltpu.sync_copy(xt_hbm.at[wid], xc_v)

        neg_inf = jnp.full((16,), -jnp.inf, dtype=jnp.float32)

        def init_body(i, carry):
            agg_v[pl.ds(i * 16, 16)] = neg_inf
            return carry

        lax.fori_loop(0, FLAT // 16, init_body, 0)

        iota = lax.iota(jnp.int32, 16)

        def chunk_body(ci, carry):
            e0 = ci * CHUNK
            pltpu.sync_copy(src_hbm.at[pl.ds(e0, CHUNK)], src_v)
            pltpu.sync_copy(dst_hbm.at[pl.ds(e0, CHUNK)], dst_v)

            def super_body(sb, c2):
                infos = []
                for u in range(SUPER):
                    off = (sb * SUPER + u) * 16
                    src = src_v[pl.ds(off, 16)]
                    dst = dst_v[pl.ds(off, 16)]
                    src4 = src * CPT
                    dst4 = dst * CPT
                    plsc.store_scatter(claim_v, [dst], iota)
                    w = plsc.load_gather(claim_v, [dst])
                    dup = w != iota
                    xvs = [plsc.load_gather(xc_v, [src4 + ch]) for ch in range(CPT)]
                    olds = [plsc.load_gather(agg_v, [dst4 + ch]) for ch in range(CPT)]
                    for ch in range(CPT):
                        plsc.store_scatter(
                            agg_v, [dst4 + ch], jnp.maximum(olds[ch], xvs[ch])
                        )
                    infos.append((dst4, xvs, dup))

                anyd = infos[0][2]
                for u in range(1, SUPER):
                    anyd = anyd | infos[u][2]

                @pl.when(jnp.max(anyd.astype(jnp.int32)) > 0)
                def _slow():
                    for dst4, xvs, dup in infos:
                        @pl.when(jnp.max(dup.astype(jnp.int32)) > 0)
                        def _fix(dst4=dst4, xvs=xvs):
                            for lane in range(16):
                                m = iota == lane
                                for ch in range(CPT):
                                    o = plsc.load_gather(agg_v, [dst4 + ch])
                                    plsc.store_scatter(
                                        agg_v,
                                        [dst4 + ch],
                                        jnp.maximum(o, xvs[ch]),
                                        mask=m,
                                    )

                return c2

            lax.fori_loop(0, NSUPER, super_body, 0)
            return carry

        lax.fori_loop(0, NCHUNK, chunk_body, 0)

        def fin_body(i, carry):
            sl = pl.ds(i * 16, 16)
            a = agg_v[sl]
            agg_v[sl] = jnp.where(a == -jnp.inf, 0.0, a) + xc_v[sl]
            return carry

        lax.fori_loop(0, FLAT // 16, fin_body, 0)
        pltpu.sync_copy(agg_v, out_hbm.at[wid])

    return k(xt, src_e, dst_e)


def _tc_mlp(h, W1, b1, W2, b2):
    BLK = 1000

    def mlp_body(h_ref, w1_ref, b1_ref, w2_ref, b2_ref, o_ref):
        hh = h_ref[...]
        z = jnp.dot(hh, w1_ref[...], preferred_element_type=jnp.float32)
        z = jnp.maximum(z + b1_ref[...], 0.0)
        o = jnp.dot(z, w2_ref[...], preferred_element_type=jnp.float32)
        o_ref[...] = o + b2_ref[...]

    return pl.pallas_call(
        mlp_body,
        grid=(N_NODES // BLK,),
        in_specs=[
            pl.BlockSpec((BLK, N_CH), lambda i: (i, 0)),
            pl.BlockSpec((N_CH, N_CH), lambda i: (0, 0)),
            pl.BlockSpec((1, N_CH), lambda i: (0, 0)),
            pl.BlockSpec((N_CH, N_CH), lambda i: (0, 0)),
            pl.BlockSpec((1, N_CH), lambda i: (0, 0)),
        ],
        out_specs=pl.BlockSpec((BLK, N_CH), lambda i: (i, 0)),
        out_shape=jax.ShapeDtypeStruct((N_NODES, N_CH), jnp.float32),
    )(h, W1, b1.reshape(1, N_CH), W2, b2.reshape(1, N_CH))


def kernel(x1, adj, W1, b1, W2, b2):
    xt = x1.reshape(N_NODES, NTILES, CPT).transpose(1, 0, 2).reshape(NTILES, FLAT)
    h32 = _sc_gather_segmax(xt, adj[0], adj[1])
    h = h32.reshape(NTILES, N_NODES, CPT).transpose(1, 0, 2).reshape(N_NODES, N_CH)
    return _tc_mlp(h, W1, b1, W2, b2)
